# baseline (speedup 1.0000x reference)
import functools

import numpy as np

import jax
import jax.numpy as jnp
from jax import lax
from jax.experimental import pallas as pl
from jax.experimental.pallas import tpu as pltpu
from jax.experimental.pallas import tpu_sc as plsc

N = 10000
E = 320000
D = 128
DP = D // 2

NC = 2
NS = 16
L = 16

NW = NC * NS
CH = 80
K = E // (NW * CH)
STAGES = (63, 62)
KSTG = max(STAGES)

ROWS_PT = N // NS
OUT_PT = (N // NS) // 8 * 8
OUT_TAIL = N - OUT_PT * NS

_PERM = np.array([32 * (c // 32) + (c % 32) // 2 + 16 * ((c % 32) % 2)
                  for c in range(D)])


def _combine_matmul_body(p_ref, w_ref, o_ref):
    o_ref[...] = lax.dot_general(
        p_ref[0] + p_ref[1], w_ref[...], (((1,), (1,)), ((), ())),
        preferred_element_type=jnp.float32)


def _tc_combine_matmul(partials, w):
    bm = 2000
    return pl.pallas_call(
        _combine_matmul_body,
        grid=(N // bm,),
        in_specs=[
            pl.BlockSpec((NC, bm, D), lambda i: (0, i, 0)),
            pl.BlockSpec((D, D), lambda i: (0, 0)),
        ],
        out_specs=pl.BlockSpec((bm, D), lambda i: (i, 0)),
        out_shape=jax.ShapeDtypeStruct((N, D), jnp.float32),
    )(partials, w)


def _sc_body(x_hbm, src_hbm, dst_hbm, val_hbm, out_hbm,
             src_all, dst_all, val_all, pk0, pk1, rf0, rf1, acc_sh,
             sem_g0, sem_g1, sem_s0, sem_s1):
    c = lax.axis_index("c")
    s = lax.axis_index("s")
    wid = c * NS + s

    pk = (pk0, pk1)
    rf = (rf0, rf1)
    sem_g = (sem_g0, sem_g1)
    sem_s = (sem_s0, sem_s1)

    zero = jnp.zeros((L,), jnp.float32)

    def z_body(i, carry):
        for r in range(D // L):
            rf0[i, pl.ds(r * L, L)] = zero
        return carry
    lax.fori_loop(0, CH, z_body, 0)

    rbase = s * OUT_PT
    for k in range(OUT_PT // CH):
        pltpu.sync_copy(rf0, acc_sh.at[pl.ds(rbase + k * CH, CH)])
    ztail = OUT_PT - (OUT_PT // CH) * CH
    if ztail:
        pltpu.sync_copy(rf0.at[pl.ds(0, ztail)],
                        acc_sh.at[pl.ds(rbase + (OUT_PT // CH) * CH, ztail)])

    @pl.when(s == 0)
    def _zero_tail():
        pltpu.sync_copy(rf0.at[pl.ds(0, OUT_TAIL)],
                        acc_sh.at[pl.ds(OUT_PT * NS, OUT_TAIL)])
    plsc.subcore_barrier()

    def g_start(chunk, b):
        pltpu.async_copy(x_hbm.at[src_all.at[pl.ds(chunk * CH, CH)]],
                         pk[b], sem_g[b])

    def g_wait(b):
        pltpu.make_async_copy(x_hbm.at[src_all.at[pl.ds(0, CH)]],
                              pk[b], sem_g[b]).wait()

    def s_start(chunk, b):
        pltpu.async_copy(rf[b], acc_sh.at[dst_all.at[pl.ds(chunk * CH, CH)]],
                         sem_s[b], add=True)

    def s_wait(b):
        pltpu.make_async_copy(rf[b], acc_sh.at[dst_all.at[pl.ds(0, CH)]],
                              sem_s[b]).wait()

    def scale(b, chunk):
        pk_ref = pk[b]
        rf_ref = rf[b]
        for g in range(CH // L):
            vals = val_all[pl.ds(chunk * CH + g * L, L)]
            for i in range(L):
                vf = jnp.full((L,), vals[i], jnp.float32)
                e = g * L + i
                for r in range(D // 32):
                    w16 = pk_ref[e, pl.ds(r * L, L)]
                    lo = lax.bitcast_convert_type(w16 << 16, jnp.float32)
                    hi = lax.bitcast_convert_type(w16 & jnp.int32(-65536),
                                                  jnp.float32)
                    rf_ref[e, pl.ds(32 * r, L)] = lo * vf
                    rf_ref[e, pl.ds(32 * r + L, L)] = hi * vf

    sbase = 0
    for S in STAGES:
        ebase = wid * (K * CH) + sbase * CH
        pltpu.sync_copy(src_hbm.at[pl.ds(ebase, S * CH)],
                        src_all.at[pl.ds(0, S * CH)])
        pltpu.sync_copy(dst_hbm.at[pl.ds(ebase, S * CH)],
                        dst_all.at[pl.ds(0, S * CH)])
        pltpu.sync_copy(val_hbm.at[pl.ds(ebase, S * CH)],
                        val_all.at[pl.ds(0, S * CH)])

        g_start(0, 0)
        g_start(1, 1)

        def chunk_body(ch, carry):
            for b in range(2):
                @pl.when(ch % 2 == b)
                def _run(b=b):
                    g_wait(b)

                    @pl.when(ch >= 2)
                    def _drain():
                        s_wait(b)
                    scale(b, ch)
                    s_start(ch, b)

                    @pl.when(ch + 2 < S)
                    def _refill():
                        g_start(ch + 2, b)
            return carry
        lax.fori_loop(0, S, chunk_body, 0)
        s_wait((S - 2) % 2)
        s_wait((S - 1) % 2)
        sbase += S

    plsc.subcore_barrier()
    orow = s * OUT_PT
    pltpu.sync_copy(acc_sh.at[pl.ds(orow, OUT_PT)],
                    out_hbm.at[c, pl.ds(orow, OUT_PT)])

    @pl.when(s == 0)
    def _copy_tail():
        pltpu.sync_copy(acc_sh.at[pl.ds(OUT_PT * NS, OUT_TAIL)],
                        out_hbm.at[c, pl.ds(OUT_PT * NS, OUT_TAIL)])


_sc_aggregate = functools.partial(
    pl.kernel,
    out_type=jax.ShapeDtypeStruct((NC, N, D), jnp.float32),
    mesh=plsc.VectorSubcoreMesh(core_axis_name="c", subcore_axis_name="s"),
    compiler_params=pltpu.CompilerParams(use_tc_tiling_on_sc=False),
    scratch_types=[
        pltpu.VMEM((KSTG * CH,), jnp.int32),
        pltpu.VMEM((KSTG * CH,), jnp.int32),
        pltpu.VMEM((KSTG * CH,), jnp.float32),
        pltpu.VMEM((CH, DP), jnp.int32),
        pltpu.VMEM((CH, DP), jnp.int32),
        pltpu.VMEM((CH, D), jnp.float32),
        pltpu.VMEM((CH, D), jnp.float32),
        pltpu.VMEM_SHARED((N, D), jnp.float32),
        pltpu.SemaphoreType.DMA,
        pltpu.SemaphoreType.DMA,
        pltpu.SemaphoreType.DMA,
        pltpu.SemaphoreType.DMA,
    ],
)(_sc_body)


def kernel(x, edge_index, edge_values, W):
    dst = edge_index[0].astype(jnp.int32)
    src = edge_index[1].astype(jnp.int32)
    xb = x.astype(jnp.bfloat16)[:, _PERM]
    xpk = lax.bitcast_convert_type(xb.reshape(N, DP, 2), jnp.int32)
    partials = _sc_aggregate(xpk, src, dst, edge_values)
    return _tc_combine_matmul(partials, W)


# speedup vs baseline: 1.1567x; 1.1567x over previous
import functools

import numpy as np

import jax
import jax.numpy as jnp
from jax import lax
from jax.experimental import pallas as pl
from jax.experimental.pallas import tpu as pltpu
from jax.experimental.pallas import tpu_sc as plsc

N = 10000
E = 320000
D = 128
DP = D // 2

NC = 2
NS = 16
L = 16

NW = NC * NS
CH = 80
K = E // (NW * CH)
STAGES = (63, 62)
KSTG = max(STAGES)

ROWS_PT = N // NS
OUT_PT = (N // NS) // 8 * 8
OUT_TAIL = N - OUT_PT * NS


def _combine_matmul_body(p_ref, w_ref, o_ref):
    o_ref[...] = lax.dot_general(
        p_ref[0] + p_ref[1], w_ref[...], (((1,), (1,)), ((), ())),
        preferred_element_type=jnp.float32)


def _tc_combine_matmul(partials, w):
    bm = 2000
    return pl.pallas_call(
        _combine_matmul_body,
        grid=(N // bm,),
        in_specs=[
            pl.BlockSpec((NC, bm, D), lambda i: (0, i, 0)),
            pl.BlockSpec((D, D), lambda i: (0, 0)),
        ],
        out_specs=pl.BlockSpec((bm, D), lambda i: (i, 0)),
        out_shape=jax.ShapeDtypeStruct((N, D), jnp.float32),
    )(partials, w)


def _sc_body(x_hbm, src_hbm, dst_hbm, val_hbm, out_hbm,
             src_all, dst_all, val_all, pk0, pk1, rf0, rf1, acc_sh,
             sem_g0, sem_g1, sem_s0, sem_s1):
    c = lax.axis_index("c")
    s = lax.axis_index("s")
    wid = c * NS + s

    pk = (pk0, pk1)
    rf = (rf0, rf1)
    sem_g = (sem_g0, sem_g1)
    sem_s = (sem_s0, sem_s1)

    zero = jnp.zeros((L,), jnp.float32)

    def z_body(i, carry):
        for r in range(D // L):
            rf0[i, pl.ds(r * L, L)] = zero
        return carry
    lax.fori_loop(0, CH, z_body, 0)

    rbase = s * OUT_PT
    for k in range(OUT_PT // CH):
        pltpu.sync_copy(rf0, acc_sh.at[pl.ds(rbase + k * CH, CH)])
    ztail = OUT_PT - (OUT_PT // CH) * CH
    if ztail:
        pltpu.sync_copy(rf0.at[pl.ds(0, ztail)],
                        acc_sh.at[pl.ds(rbase + (OUT_PT // CH) * CH, ztail)])

    @pl.when(s == 0)
    def _zero_tail():
        pltpu.sync_copy(rf0.at[pl.ds(0, OUT_TAIL)],
                        acc_sh.at[pl.ds(OUT_PT * NS, OUT_TAIL)])
    plsc.subcore_barrier()

    def g_start(chunk, b):
        pltpu.async_copy(x_hbm.at[src_all.at[pl.ds(chunk * CH, CH)]],
                         pk[b], sem_g[b])

    def g_wait(b):
        pltpu.make_async_copy(x_hbm.at[src_all.at[pl.ds(0, CH)]],
                              pk[b], sem_g[b]).wait()

    def s_start(chunk, b):
        pltpu.async_copy(rf[b], acc_sh.at[dst_all.at[pl.ds(chunk * CH, CH)]],
                         sem_s[b], add=True)

    def s_wait(b):
        pltpu.make_async_copy(rf[b], acc_sh.at[dst_all.at[pl.ds(0, CH)]],
                              sem_s[b]).wait()

    def scale(b, chunk):
        pk_ref = pk[b]
        rf_ref = rf[b]
        for g in range(CH // L):
            vals = val_all[pl.ds(chunk * CH + g * L, L)]
            for i in range(L):
                vf = jnp.full((L,), vals[i], jnp.float32)
                e = g * L + i
                for r in range(D // 32):
                    w16 = pk_ref[e, pl.ds(r * L, L)]
                    lo = lax.bitcast_convert_type(w16 << 16, jnp.float32)
                    hi = lax.bitcast_convert_type(w16 & jnp.int32(-65536),
                                                  jnp.float32)
                    rf_ref[e, pl.ds(32 * r, L)] = lo * vf
                    rf_ref[e, pl.ds(32 * r + L, L)] = hi * vf

    sbase = 0
    for S in STAGES:
        ebase = wid * (K * CH) + sbase * CH
        pltpu.sync_copy(src_hbm.at[pl.ds(ebase, S * CH)],
                        src_all.at[pl.ds(0, S * CH)])
        pltpu.sync_copy(dst_hbm.at[pl.ds(ebase, S * CH)],
                        dst_all.at[pl.ds(0, S * CH)])
        pltpu.sync_copy(val_hbm.at[pl.ds(ebase, S * CH)],
                        val_all.at[pl.ds(0, S * CH)])

        g_start(0, 0)
        g_start(1, 1)

        def chunk_body(ch, carry):
            for b in range(2):
                @pl.when(ch % 2 == b)
                def _run(b=b):
                    g_wait(b)

                    @pl.when(ch >= 2)
                    def _drain():
                        s_wait(b)
                    scale(b, ch)
                    s_start(ch, b)

                    @pl.when(ch + 2 < S)
                    def _refill():
                        g_start(ch + 2, b)
            return carry
        lax.fori_loop(0, S, chunk_body, 0)
        s_wait((S - 2) % 2)
        s_wait((S - 1) % 2)
        sbase += S

    plsc.subcore_barrier()
    orow = s * OUT_PT
    pltpu.sync_copy(acc_sh.at[pl.ds(orow, OUT_PT)],
                    out_hbm.at[c, pl.ds(orow, OUT_PT)])

    @pl.when(s == 0)
    def _copy_tail():
        pltpu.sync_copy(acc_sh.at[pl.ds(OUT_PT * NS, OUT_TAIL)],
                        out_hbm.at[c, pl.ds(OUT_PT * NS, OUT_TAIL)])


_sc_aggregate = functools.partial(
    pl.kernel,
    out_type=jax.ShapeDtypeStruct((NC, N, D), jnp.float32),
    mesh=plsc.VectorSubcoreMesh(core_axis_name="c", subcore_axis_name="s"),
    compiler_params=pltpu.CompilerParams(use_tc_tiling_on_sc=False),
    scratch_types=[
        pltpu.VMEM((KSTG * CH,), jnp.int32),
        pltpu.VMEM((KSTG * CH,), jnp.int32),
        pltpu.VMEM((KSTG * CH,), jnp.float32),
        pltpu.VMEM((CH, DP), jnp.int32),
        pltpu.VMEM((CH, DP), jnp.int32),
        pltpu.VMEM((CH, D), jnp.float32),
        pltpu.VMEM((CH, D), jnp.float32),
        pltpu.VMEM_SHARED((N, D), jnp.float32),
        pltpu.SemaphoreType.DMA,
        pltpu.SemaphoreType.DMA,
        pltpu.SemaphoreType.DMA,
        pltpu.SemaphoreType.DMA,
    ],
)(_sc_body)


def kernel(x, edge_index, edge_values, W):
    dst = edge_index[0].astype(jnp.int32)
    src = edge_index[1].astype(jnp.int32)
    xbits = lax.bitcast_convert_type(x.astype(jnp.bfloat16), jnp.uint16)
    halves = xbits.reshape(N, D // 32, 2, L).astype(jnp.int32)
    xpk = (halves[:, :, 0, :] | (halves[:, :, 1, :] << 16)).reshape(N, DP)
    partials = _sc_aggregate(xpk, src, dst, edge_values)
    return _tc_combine_matmul(partials, W)
